# Initial kernel scaffold; baseline (speedup 1.0000x reference)
import jax
import jax.numpy as jnp
from jax.experimental import pallas as pl


def kernel(x, e2e, batch):
    raise NotImplementedError("write your pallas kernel here")


# speedup vs baseline: 8.1699x; 8.1699x over previous
import functools

import jax
import jax.numpy as jnp
from jax.experimental import pallas as pl
from jax.experimental.pallas import tpu as pltpu
from jax.experimental.pallas import tpu_sc as plsc

E = 160000
C = 256
NG = 8
K = 16000
NKEPT = NG * K

ROWS_PER_BLK = 1024
NBLK = (E + ROWS_PER_BLK - 1) // ROWS_PER_BLK


def _keys_body(x_ref, k_ref):
    xx = x_ref[...]
    a = xx * xx
    base = a[:, :128] + a[:, 128:]
    acc = base[:, 0:8]
    for w in range(1, 16):
        acc = acc + base[:, 8 * w:8 * w + 8]
    acc = acc[:, :4] + acc[:, 4:]
    acc = acc[:, :2] + acc[:, 2:]
    s2 = acc[:, 0] + acc[:, 1]
    score = jnp.sqrt(s2)
    bits = jax.lax.bitcast_convert_type(score, jnp.int32)
    k_ref[...] = jnp.int32(0x7FFFFFFF) - bits


def _score_keys(x):
    return pl.pallas_call(
        _keys_body,
        grid=(NBLK,),
        in_specs=[pl.BlockSpec((ROWS_PER_BLK, C), lambda i: (i, 0))],
        out_specs=pl.BlockSpec((ROWS_PER_BLK,), lambda i: (i,)),
        out_shape=jax.ShapeDtypeStruct((E,), jnp.int32),
    )(x)


NT = 16
CH = E // NT
NKT = NKEPT // NT
I16 = lambda: jax.lax.iota(jnp.int32, 16)


def _sc_sort_kernel(keys_hbm, batch_hbm, kept_hbm,
                    keys_sp, bufA_v, bufB_v, hist_sp, cnt_sp,
                    kbuf, vbuf, dbuf, gbuf, dbuf8,
                    histallv, histv, offsv, gstartv, gcntv):
    cid = jax.lax.axis_index("c")
    sid = jax.lax.axis_index("s")

    @pl.when(cid == 0)
    def _():
        t = sid
        base = t * CH

        pltpu.sync_copy(keys_hbm.at[pl.ds(base, CH)], kbuf)
        pltpu.sync_copy(batch_hbm.at[pl.ds(base, CH)], dbuf)

        def init_body(i, c):
            vbuf[pl.ds(i * 16, 16)] = base + i * 16 + I16()
            return c
        jax.lax.fori_loop(0, CH // 16, init_body, 0)

        def cnt_body(i, acc):
            b = dbuf[pl.ds(i * 16, 16)]
            upd = acc
            for g in range(NG):
                n = jnp.sum(jnp.where(b == g, 1, 0).astype(jnp.int32))
                upd = upd + jnp.where(I16() == g, n, 0)
            return upd
        mycnt = jax.lax.fori_loop(0, CH // 16, cnt_body,
                                  jnp.zeros((16,), jnp.int32))
        histv[...] = mycnt
        pltpu.sync_copy(histv, cnt_sp.at[pl.ds(t * 16, 16)])
        pltpu.sync_copy(kbuf, keys_sp.at[pl.ds(base, CH)])
        pltpu.sync_copy(vbuf, bufA_v.at[pl.ds(base, CH)])
        plsc.subcore_barrier()

        pltpu.sync_copy(cnt_sp, histallv)

        def gsum_body(i, acc):
            return acc + histallv[pl.ds(i * 16, 16)]
        gcnt = jax.lax.fori_loop(0, NT, gsum_body, jnp.zeros((16,), jnp.int32))
        gstart = plsc.cumsum(gcnt) - gcnt
        gcntv[...] = gcnt
        gstartv[...] = gstart
        starts = [jnp.sum(jnp.where(I16() == j, gstart, 0)) for j in range(1, NG)]

        bufs = [bufA_v, bufB_v]
        for p in range(9):
            src_v = bufs[p % 2]
            dst_v = bufs[(p + 1) % 2]
            pltpu.sync_copy(src_v.at[pl.ds(base, CH)], vbuf)
            if p < 8:
                pltpu.sync_copy(keys_sp.at[vbuf], kbuf)

            if p < 8:
                def digit_fn(i):
                    k = kbuf[pl.ds(i * 16, 16)]
                    return jax.lax.shift_right_logical(k, p * 4) & 15
            else:
                def digit_fn(i):
                    v = vbuf[pl.ds(i * 16, 16)]
                    g = jnp.zeros((16,), jnp.int32)
                    for s in starts:
                        g = g + jnp.where(v >= s, 1, 0)
                    return g

            histv[...] = jnp.zeros((16,), jnp.int32)

            def hist_body(i, c):
                d = digit_fn(i)
                h = plsc.load_gather(histv, [d])
                cnt, last = plsc.scan_count(d)
                plsc.store_scatter(histv, [d], h + cnt, mask=last)
                return c
            jax.lax.fori_loop(0, CH // 16, hist_body, 0)
            pltpu.sync_copy(histv, hist_sp.at[pl.ds(t * 16, 16)])
            plsc.subcore_barrier()

            pltpu.sync_copy(hist_sp, histallv)
            running = jnp.int32(0)
            offv = jnp.zeros((16,), jnp.int32)
            for b in range(16):
                binvec = plsc.load_gather(histallv, [I16() * 16 + b])
                tot = jnp.sum(binvec)
                before = jnp.sum(jnp.where(I16() < t, binvec, 0))
                offv = offv + jnp.where(I16() == b, running + before, 0)
                running = running + tot
            offsv[...] = offv

            def perm_body(i, c):
                d = digit_fn(i)
                bs = plsc.load_gather(offsv, [d])
                cnt, last = plsc.scan_count(d)
                plsc.store_scatter(offsv, [d], bs + cnt, mask=last)
                dbuf[pl.ds(i * 16, 16)] = bs + cnt - 1
                return c
            jax.lax.fori_loop(0, CH // 16, perm_body, 0)
            pltpu.sync_copy(vbuf, dst_v.at[dbuf])
            plsc.subcore_barrier()

        g = sid >> 1
        half = sid & 1
        gv = jnp.zeros((16,), jnp.int32) + g
        sg = plsc.load_gather(gstartv, [gv])
        cg = plsc.load_gather(gcntv, [gv])

        def ext1_body(i, c):
            j = half * NKT + i * 16 + I16()
            src = sg + j
            in1 = j < cg
            dbuf8[pl.ds(i * 16, 16)] = jnp.where(in1, src, 0)
            return c
        jax.lax.fori_loop(0, NKT // 16, ext1_body, 0)
        pltpu.sync_copy(bufB_v.at[dbuf8], gbuf)

        def ext2_body(i, c):
            j = half * NKT + i * 16 + I16()
            in1 = j < cg
            fillrel = j - cg
            fillidx = jnp.where(fillrel < sg, fillrel, fillrel + cg)
            out = jnp.where(in1, gbuf[pl.ds(i * 16, 16)], fillidx)
            kbuf[pl.ds(i * 16, 16)] = out
            return c
        jax.lax.fori_loop(0, NKT // 16, ext2_body, 0)
        pltpu.sync_copy(kbuf.at[pl.ds(0, NKT)], kept_hbm.at[pl.ds(t * NKT, NKT)])


def _sc_sort(keys, batch):
    mesh = plsc.VectorSubcoreMesh(core_axis_name="c", subcore_axis_name="s")
    f = functools.partial(
        pl.kernel,
        out_type=jax.ShapeDtypeStruct((NKEPT,), jnp.int32),
        mesh=mesh,
        compiler_params=pltpu.CompilerParams(needs_layout_passes=False),
        scratch_types=[
            pltpu.VMEM_SHARED((E,), jnp.int32),
            pltpu.VMEM_SHARED((E,), jnp.int32),
            pltpu.VMEM_SHARED((E,), jnp.int32),
            pltpu.VMEM_SHARED((NT * 16,), jnp.int32),
            pltpu.VMEM_SHARED((NT * 16,), jnp.int32),
            pltpu.VMEM((CH,), jnp.int32),
            pltpu.VMEM((CH,), jnp.int32),
            pltpu.VMEM((CH,), jnp.int32),
            pltpu.VMEM((NKT,), jnp.int32),
            pltpu.VMEM((NKT,), jnp.int32),
            pltpu.VMEM((NT * 16,), jnp.int32),
            pltpu.VMEM((16,), jnp.int32),
            pltpu.VMEM((16,), jnp.int32),
            pltpu.VMEM((16,), jnp.int32),
            pltpu.VMEM((16,), jnp.int32),
        ],
    )(_sc_sort_kernel)
    return f(keys, batch)


NW = 32
RW = NKEPT // NW
RCH = 80
SUBK = NKEPT // 16
HB = RW * 2


def _sc_gather_kernel(kept_hbm, x_hbm, e2e_hbm, batch_hbm,
                      xout_hbm, eout_hbm, bout_hbm,
                      o2n_sp, kbuf, kb8, rowbuf, ebuf, ibuf, mbuf, obuf, bbuf):
    cid = jax.lax.axis_index("c")
    sid = jax.lax.axis_index("s")
    wid = cid * 16 + sid
    base = wid * RW

    def fill_body(i, c):
        mbuf[pl.ds(i * 16, 16)] = jnp.full((16,), -1, jnp.int32)
        return c
    jax.lax.fori_loop(0, HB // 16, fill_body, 0)
    pltpu.sync_copy(mbuf, o2n_sp.at[pl.ds(sid * (E // NT), HB)])
    pltpu.sync_copy(mbuf.at[pl.ds(0, E // NT - HB)],
                    o2n_sp.at[pl.ds(sid * (E // NT) + HB, E // NT - HB)])
    plsc.subcore_barrier()

    pltpu.sync_copy(kept_hbm.at[pl.ds(sid * SUBK, SUBK)], kb8)

    def nid_body(i, c):
        mbuf[pl.ds(i * 16, 16)] = sid * SUBK + i * 16 + I16()
        return c
    jax.lax.fori_loop(0, SUBK // 16, nid_body, 0)
    pltpu.sync_copy(mbuf, o2n_sp.at[kb8])
    plsc.subcore_barrier()

    pltpu.sync_copy(kept_hbm.at[pl.ds(base, RW)], kbuf)

    pltpu.sync_copy(batch_hbm.at[kbuf], bbuf)
    pltpu.sync_copy(bbuf, bout_hbm.at[pl.ds(base, RW)])

    for ci in range(RW // RCH):
        pltpu.sync_copy(x_hbm.at[kbuf.at[pl.ds(ci * RCH, RCH)]], rowbuf)
        pltpu.sync_copy(rowbuf, xout_hbm.at[pl.ds(base + ci * RCH, RCH)])

    for h in range(2):
        r0 = h * (RW // 2)

        def fidx_body(i, c):
            f = i * 16 + I16()
            kv = plsc.load_gather(kbuf, [r0 + jax.lax.shift_right_logical(f, 2)])
            ibuf[pl.ds(i * 16, 16)] = kv * 4 + (f & 3)
            return c
        jax.lax.fori_loop(0, HB // 16, fidx_body, 0)
        pltpu.sync_copy(e2e_hbm.at[ibuf], ebuf)

        def eidx_body(i, c):
            e = ebuf[pl.ds(i * 16, 16)]
            ibuf[pl.ds(i * 16, 16)] = jnp.maximum(e, 0)
            return c
        jax.lax.fori_loop(0, HB // 16, eidx_body, 0)
        pltpu.sync_copy(o2n_sp.at[ibuf], mbuf)

        def emap_body(i, c):
            f = i * 16 + I16()
            m = mbuf[pl.ds(i * 16, 16)]
            self_id = base + r0 + jax.lax.shift_right_logical(f, 2)
            obuf[pl.ds(i * 16, 16)] = jnp.where(m < 0, self_id, m)
            return c
        jax.lax.fori_loop(0, HB // 16, emap_body, 0)
        pltpu.sync_copy(obuf, eout_hbm.at[pl.ds(base * 4 + h * HB, HB)])


def _sc_gather(kept, x, e2e_flat, batch):
    mesh = plsc.VectorSubcoreMesh(core_axis_name="c", subcore_axis_name="s")
    f = functools.partial(
        pl.kernel,
        out_type=[
            jax.ShapeDtypeStruct((NKEPT, C), jnp.float32),
            jax.ShapeDtypeStruct((NKEPT * 4,), jnp.int32),
            jax.ShapeDtypeStruct((NKEPT,), jnp.int32),
        ],
        mesh=mesh,
        compiler_params=pltpu.CompilerParams(needs_layout_passes=False),
        scratch_types=[
            pltpu.VMEM_SHARED((E,), jnp.int32),
            pltpu.VMEM((RW,), jnp.int32),
            pltpu.VMEM((SUBK,), jnp.int32),
            pltpu.VMEM((RCH, C), jnp.float32),
            pltpu.VMEM((HB,), jnp.int32),
            pltpu.VMEM((HB,), jnp.int32),
            pltpu.VMEM((HB,), jnp.int32),
            pltpu.VMEM((HB,), jnp.int32),
            pltpu.VMEM((RW,), jnp.int32),
        ],
    )(_sc_gather_kernel)
    return f(kept, x, e2e_flat, batch)


def kernel(x, e2e, batch):
    keys = _score_keys(x)
    kept = _sc_sort(keys, batch)
    xo, eo, bo = _sc_gather(kept, x, e2e.reshape(E * 4), batch)
    return xo, eo.reshape(NKEPT, 4), bo
